# baseline (speedup 1.0000x reference)
import functools
import math

import numpy as np
import jax
import jax.numpy as jnp
from jax.experimental import pallas as pl
from jax.experimental.pallas import tpu as pltpu

_NEG = -1e30

_CT_CACHE = {}


def _rotl32(x, d):
    return ((x << np.uint32(d)) | (x >> np.uint32(32 - d))).astype(np.uint32)


def _threefry2x32(k0, k1, x0, x1):
    rotations = ((13, 15, 26, 6), (17, 29, 16, 24))
    ks = (np.uint32(k0), np.uint32(k1),
          np.uint32(k0 ^ k1 ^ np.uint32(0x1BD11BDA)))
    x0 = (x0 + ks[0]).astype(np.uint32)
    x1 = (x1 + ks[1]).astype(np.uint32)
    for i in range(5):
        for r in rotations[i % 2]:
            x0 = (x0 + x1).astype(np.uint32)
            x1 = _rotl32(x1, r)
            x1 = (x1 ^ x0).astype(np.uint32)
        x0 = (x0 + ks[(i + 1) % 3]).astype(np.uint32)
        x1 = (x1 + ks[(i + 2) % 3] + np.uint32(i + 1)).astype(np.uint32)
    return x0, x1


def _sample_indices(seed, L_Q, U_part, L_K):
    assert L_K & (L_K - 1) == 0 and L_K <= (1 << 16)
    hi = np.zeros(2, np.uint32)
    lo = np.arange(2, dtype=np.uint32)
    b1, b2 = _threefry2x32(np.uint32(0), np.uint32(seed), hi, lo)
    k0, k1 = b1[1], b2[1]
    n = L_Q * U_part
    b1, b2 = _threefry2x32(k0, k1, np.zeros(n, np.uint32),
                           np.arange(n, dtype=np.uint32))
    bits = (b1 ^ b2).astype(np.uint32)
    return (bits % np.uint32(L_K)).astype(np.int32).reshape(L_Q, U_part)


def _count_matrix_t(L_Q, L_K, U_part):
    key = (L_Q, L_K, U_part)
    if key not in _CT_CACHE:
        idx = _sample_indices(1234, L_Q, U_part, L_K)
        ct = np.zeros((L_K, L_Q), np.float32)
        np.add.at(ct, (idx.reshape(-1),
                       np.repeat(np.arange(L_Q), U_part)), 1.0)
        mask_add = np.where(ct > 0, 0.0, _NEG).astype(np.float32)
        _CT_CACHE[key] = (ct, mask_add)
    return _CT_CACHE[key]


def _measure_topk_body(u, u_pad, n_chunks, prec, q_ref, k_ref, ct_ref,
                       msk_ref, idx_ref, m_scr):
    B, L = m_scr.shape
    blk = L // n_chunks
    b = pl.program_id(0)
    k = k_ref[0]
    for lb in range(n_chunks):
        q_blk = q_ref[0, lb * blk:(lb + 1) * blk, :]
        s_t = jax.lax.dot_general(
            k, q_blk, (((1,), (1,)), ((), ())),
            preferred_element_type=jnp.float32, precision=prec)
        c_t = ct_ref[:, lb * blk:(lb + 1) * blk].astype(jnp.float32)
        a_t = msk_ref[:, lb * blk:(lb + 1) * blk].astype(jnp.float32)
        smax = jnp.max(s_t + a_t, axis=0, keepdims=True)
        ssum = jnp.sum(s_t * c_t, axis=0, keepdims=True)
        m_scr[pl.ds(b, 1), lb * blk:(lb + 1) * blk] = smax - ssum / L

    @pl.when(b == B - 1)
    def _():
        m = m_scr[...]
        iota = jax.lax.broadcasted_iota(jnp.int32, (B, L), 1)
        acc_iota = jax.lax.broadcasted_iota(jnp.int32, (B, u_pad), 1)
        acc = jnp.zeros((B, u_pad), jnp.int32)
        for i in range(u):
            v, ix = m, iota
            w = L
            while w > 128:
                h = w // 2
                lv, rv = v[:, :h], v[:, h:]
                li, ri = ix[:, :h], ix[:, h:]
                take = rv > lv
                v = jnp.where(take, rv, lv)
                ix = jnp.where(take, ri, li)
                w = h
            mx = jnp.max(v, axis=1, keepdims=True)
            idx = jnp.min(jnp.where(v == mx, ix, L), axis=1,
                          keepdims=True)
            acc = jnp.where(acc_iota == i, idx, acc)
            m = jnp.where(iota == idx, _NEG, m)
        idx_ref[:, 0, :] = acc


def _attend_body(u, prec, q_ref, k_ref, v_ref, idx_ref, o_ref,
                 qr_ref, upd_ref):
    L, D = q_ref.shape[1], q_ref.shape[2]
    k = k_ref[0]
    v = v_ref[0]
    for i in range(u):
        qr_ref[pl.ds(i, 1), :] = q_ref[0, pl.ds(idx_ref[0, 0, i], 1), :]
    qr = qr_ref[...]
    st = jax.lax.dot_general(
        qr, k, (((1,), (1,)), ((), ())),
        preferred_element_type=jnp.float32, precision=prec)
    st = st * (1.0 / math.sqrt(D))
    smx = jnp.max(st, axis=1, keepdims=True)
    p = jnp.exp(st - smx)
    p = p / jnp.sum(p, axis=1, keepdims=True)
    upd_ref[...] = jax.lax.dot_general(
        p, v, (((1,), (0,)), ((), ())),
        preferred_element_type=jnp.float32, precision=prec)

    vmean = jnp.sum(v, axis=0, keepdims=True) / L
    o_ref[0] = jnp.broadcast_to(vmean, (L, D))
    for i in range(u):
        o_ref[0, pl.ds(idx_ref[0, 0, i], 1), :] = upd_ref[pl.ds(i, 1), :]


@jax.jit
def kernel(queries, keys, values):
    B, L_Q, D = queries.shape
    _, L_K, _ = keys.shape
    factor = 5
    U_part = min(factor * int(np.ceil(np.log(L_K))), L_K)
    u = min(factor * int(np.ceil(np.log(L_Q))), L_Q)
    u_pad = max(8 * ((u + 7) // 8), 64)

    ct_np, mask_np = _count_matrix_t(L_Q, L_K, U_part)
    ct = jnp.asarray(ct_np, dtype=jnp.bfloat16)
    msk = jnp.asarray(mask_np, dtype=jnp.bfloat16)
    n_chunks = 8
    prec = jax.lax.Precision.DEFAULT

    idx = pl.pallas_call(
        functools.partial(_measure_topk_body, u, u_pad, n_chunks, prec),
        grid=(B,),
        in_specs=[
            pl.BlockSpec((1, L_Q, D), lambda b: (b, 0, 0)),
            pl.BlockSpec((1, L_K, D), lambda b: (b, 0, 0)),
            pl.BlockSpec((L_K, L_Q), lambda b: (0, 0)),
            pl.BlockSpec((L_K, L_Q), lambda b: (0, 0)),
        ],
        out_specs=pl.BlockSpec((B, 1, u_pad), lambda b: (0, 0, 0)),
        out_shape=jax.ShapeDtypeStruct((B, 1, u_pad), jnp.int32),
        scratch_shapes=[pltpu.VMEM((B, L_Q), jnp.float32)],
    )(queries, keys, ct, msk)

    out = pl.pallas_call(
        functools.partial(_attend_body, u, prec),
        grid=(B,),
        in_specs=[
            pl.BlockSpec((1, L_Q, D), lambda b: (b, 0, 0)),
            pl.BlockSpec((1, L_K, D), lambda b: (b, 0, 0)),
            pl.BlockSpec((1, L_K, D), lambda b: (b, 0, 0)),
            pl.BlockSpec((1, 1, u_pad), lambda b: (b, 0, 0),
                         memory_space=pltpu.SMEM),
        ],
        out_specs=pl.BlockSpec((1, L_Q, D), lambda b: (b, 0, 0)),
        out_shape=jax.ShapeDtypeStruct((B, L_Q, D), jnp.float32),
        scratch_shapes=[
            pltpu.VMEM((u, D), jnp.float32),
            pltpu.VMEM((u, D), jnp.float32),
        ],
    )(queries, keys, values, idx)
    return out


# speedup vs baseline: 1.1811x; 1.1811x over previous
import functools
import math

import numpy as np
import jax
import jax.numpy as jnp
from jax.experimental import pallas as pl
from jax.experimental.pallas import tpu as pltpu

_NEG = -1e30

_CT_CACHE = {}


def _rotl32(x, d):
    return ((x << np.uint32(d)) | (x >> np.uint32(32 - d))).astype(np.uint32)


def _threefry2x32(k0, k1, x0, x1):
    rotations = ((13, 15, 26, 6), (17, 29, 16, 24))
    ks = (np.uint32(k0), np.uint32(k1),
          np.uint32(k0 ^ k1 ^ np.uint32(0x1BD11BDA)))
    x0 = (x0 + ks[0]).astype(np.uint32)
    x1 = (x1 + ks[1]).astype(np.uint32)
    for i in range(5):
        for r in rotations[i % 2]:
            x0 = (x0 + x1).astype(np.uint32)
            x1 = _rotl32(x1, r)
            x1 = (x1 ^ x0).astype(np.uint32)
        x0 = (x0 + ks[(i + 1) % 3]).astype(np.uint32)
        x1 = (x1 + ks[(i + 2) % 3] + np.uint32(i + 1)).astype(np.uint32)
    return x0, x1


def _sample_indices(seed, L_Q, U_part, L_K):
    assert L_K & (L_K - 1) == 0 and L_K <= (1 << 16)
    hi = np.zeros(2, np.uint32)
    lo = np.arange(2, dtype=np.uint32)
    b1, b2 = _threefry2x32(np.uint32(0), np.uint32(seed), hi, lo)
    k0, k1 = b1[1], b2[1]
    n = L_Q * U_part
    b1, b2 = _threefry2x32(k0, k1, np.zeros(n, np.uint32),
                           np.arange(n, dtype=np.uint32))
    bits = (b1 ^ b2).astype(np.uint32)
    return (bits % np.uint32(L_K)).astype(np.int32).reshape(L_Q, U_part)


def _count_matrix_t(L_Q, L_K, U_part):
    key = (L_Q, L_K, U_part)
    if key not in _CT_CACHE:
        idx = _sample_indices(1234, L_Q, U_part, L_K)
        ct = np.zeros((L_K, L_Q), np.float32)
        np.add.at(ct, (idx.reshape(-1),
                       np.repeat(np.arange(L_Q), U_part)), 1.0)
        mask_add = np.where(ct > 0, 0.0, _NEG).astype(np.float32)
        _CT_CACHE[key] = (ct, mask_add)
    return _CT_CACHE[key]


def _measure_topk_body(u, u_pad, n_chunks, prec, q_ref, k_ref, ct_ref,
                       msk_ref, idx_ref, m_scr):
    B, L = m_scr.shape
    blk = L // n_chunks
    b = pl.program_id(0)
    k = k_ref[0]
    for lb in range(n_chunks):
        q_blk = q_ref[0, lb * blk:(lb + 1) * blk, :]
        s_t = jax.lax.dot_general(
            k, q_blk, (((1,), (1,)), ((), ())),
            preferred_element_type=jnp.float32, precision=prec)
        c_t = ct_ref[:, lb * blk:(lb + 1) * blk]
        a_t = msk_ref[:, lb * blk:(lb + 1) * blk]
        smax = jnp.max(s_t + a_t, axis=0, keepdims=True)
        ssum = jnp.sum(s_t * c_t, axis=0, keepdims=True)
        m_scr[pl.ds(b, 1), lb * blk:(lb + 1) * blk] = smax - ssum / L

    @pl.when(b == B - 1)
    def _():
        m = m_scr[...]
        iota = jax.lax.broadcasted_iota(jnp.int32, (B, L), 1)
        acc_iota = jax.lax.broadcasted_iota(jnp.int32, (B, u_pad), 1)
        acc = jnp.zeros((B, u_pad), jnp.int32)
        for i in range(u):
            v, ix = m, iota
            w = L
            while w > 128:
                h = w // 2
                lv, rv = v[:, :h], v[:, h:]
                li, ri = ix[:, :h], ix[:, h:]
                take = rv > lv
                v = jnp.where(take, rv, lv)
                ix = jnp.where(take, ri, li)
                w = h
            mx = jnp.max(v, axis=1, keepdims=True)
            idx = jnp.min(jnp.where(v == mx, ix, L), axis=1,
                          keepdims=True)
            acc = jnp.where(acc_iota == i, idx, acc)
            m = jnp.where(iota == idx, _NEG, m)
        idx_ref[:, 0, :] = acc


def _attend_body(u, prec, q_ref, k_ref, v_ref, idx_ref, o_ref,
                 qr_ref, upd_ref):
    L, D = q_ref.shape[1], q_ref.shape[2]
    k = k_ref[0]
    v = v_ref[0]
    for i in range(u):
        qr_ref[pl.ds(i, 1), :] = q_ref[0, pl.ds(idx_ref[0, 0, i], 1), :]
    qr = qr_ref[...]
    st = jax.lax.dot_general(
        qr, k, (((1,), (1,)), ((), ())),
        preferred_element_type=jnp.float32, precision=prec)
    st = st * (1.0 / math.sqrt(D))
    smx = jnp.max(st, axis=1, keepdims=True)
    p = jnp.exp(st - smx)
    p = p / jnp.sum(p, axis=1, keepdims=True)
    upd_ref[...] = jax.lax.dot_general(
        p, v, (((1,), (0,)), ((), ())),
        preferred_element_type=jnp.float32, precision=prec)

    vmean = jnp.sum(v, axis=0, keepdims=True) / L
    o_ref[0] = jnp.broadcast_to(vmean, (L, D))
    for i in range(u):
        o_ref[0, pl.ds(idx_ref[0, 0, i], 1), :] = upd_ref[pl.ds(i, 1), :]


@jax.jit
def kernel(queries, keys, values):
    B, L_Q, D = queries.shape
    _, L_K, _ = keys.shape
    factor = 5
    U_part = min(factor * int(np.ceil(np.log(L_K))), L_K)
    u = min(factor * int(np.ceil(np.log(L_Q))), L_Q)
    u_pad = max(8 * ((u + 7) // 8), 64)

    ct_np, mask_np = _count_matrix_t(L_Q, L_K, U_part)
    ct = jnp.asarray(ct_np)
    msk = jnp.asarray(mask_np)
    n_chunks = 4
    prec = jax.lax.Precision.DEFAULT

    idx = pl.pallas_call(
        functools.partial(_measure_topk_body, u, u_pad, n_chunks, prec),
        grid=(B,),
        in_specs=[
            pl.BlockSpec((1, L_Q, D), lambda b: (b, 0, 0)),
            pl.BlockSpec((1, L_K, D), lambda b: (b, 0, 0)),
            pl.BlockSpec((L_K, L_Q), lambda b: (0, 0)),
            pl.BlockSpec((L_K, L_Q), lambda b: (0, 0)),
        ],
        out_specs=pl.BlockSpec((B, 1, u_pad), lambda b: (0, 0, 0)),
        out_shape=jax.ShapeDtypeStruct((B, 1, u_pad), jnp.int32),
        scratch_shapes=[pltpu.VMEM((B, L_Q), jnp.float32)],
    )(queries, keys, ct, msk)

    out = pl.pallas_call(
        functools.partial(_attend_body, u, prec),
        grid=(B,),
        in_specs=[
            pl.BlockSpec((1, L_Q, D), lambda b: (b, 0, 0)),
            pl.BlockSpec((1, L_K, D), lambda b: (b, 0, 0)),
            pl.BlockSpec((1, L_K, D), lambda b: (b, 0, 0)),
            pl.BlockSpec((1, 1, u_pad), lambda b: (b, 0, 0),
                         memory_space=pltpu.SMEM),
        ],
        out_specs=pl.BlockSpec((1, L_Q, D), lambda b: (b, 0, 0)),
        out_shape=jax.ShapeDtypeStruct((B, L_Q, D), jnp.float32),
        scratch_shapes=[
            pltpu.VMEM((u, D), jnp.float32),
            pltpu.VMEM((u, D), jnp.float32),
        ],
    )(queries, keys, values, idx)
    return out
